# baseline (speedup 1.0000x reference)
import functools

import jax
import jax.numpy as jnp
from jax import lax
from jax.experimental import pallas as pl
from jax.experimental.pallas import tpu as pltpu
from jax.experimental.pallas import tpu_sc as plsc

B = 16384
NCAT = 26
NNUM = 13
D = 32
CARD = 100000
NW = 32
CB = B // NW
L = 16

_mesh = plsc.VectorSubcoreMesh(core_axis_name="c", subcore_axis_name="s")


@functools.partial(
    pl.kernel,
    out_type=jax.ShapeDtypeStruct((B, NNUM + NCAT, D), jnp.float32),
    mesh=_mesh,
    compiler_params=pltpu.CompilerParams(
        needs_layout_passes=False, use_tc_tiling_on_sc=False),
    scratch_types=[
        pltpu.VMEM((CB,), jnp.int32),
        pltpu.VMEM((CB, D), jnp.float32),
        pltpu.VMEM((CB * NNUM,), jnp.float32),
        pltpu.VMEM((NNUM, D), jnp.float32),
        pltpu.VMEM((NNUM, D), jnp.float32),
        pltpu.VMEM((NCAT, D), jnp.float32),
        pltpu.SemaphoreType.DMA,
    ],
)
def _emb(x_num, xcT, W_num, b_num, table, cat_bias, out,
         idx_v, rows_v, xnum_v, Wv, Bv, Cv, sem):
    c = lax.axis_index("c")
    s = lax.axis_index("s")
    wid = s * 2 + c
    b0 = wid * CB

    pltpu.sync_copy(W_num, Wv)
    pltpu.sync_copy(b_num, Bv)
    pltpu.sync_copy(cat_bias, Cv)
    pltpu.sync_copy(x_num.at[pl.ds(b0 * NNUM, CB * NNUM)], xnum_v)

    def num_i(i, carry):
        w_lo = Wv[i, pl.ds(0, L)]
        w_hi = Wv[i, pl.ds(L, L)]
        bl = Bv[i, pl.ds(0, L)]
        bh = Bv[i, pl.ds(L, L)]
        def num_b(bb, carry2):
            iv = jnp.full((L,), bb * NNUM + i, jnp.int32)
            v = plsc.load_gather(xnum_v, [iv])
            rows_v[bb, pl.ds(0, L)] = v * w_lo + bl
            rows_v[bb, pl.ds(L, L)] = v * w_hi + bh
            return carry2

        lax.fori_loop(0, CB, num_b, 0)
        pltpu.sync_copy(rows_v, out.at[pl.ds(b0, CB), i])
        return carry

    lax.fori_loop(0, NNUM, num_i, 0)

    def cat_j(j, carry):
        pltpu.sync_copy(xcT.at[j, pl.ds(b0, CB)], idx_v)
        off = jnp.full((L,), j * CARD, jnp.int32)
        for p in range(CB // L):
            idx_v[pl.ds(p * L, L)] = idx_v[pl.ds(p * L, L)] + off
        pltpu.async_copy(table.at[idx_v], rows_v, sem).wait()
        cl = Cv[j, pl.ds(0, L)]
        ch = Cv[j, pl.ds(L, L)]

        def bias_r(r, carry2):
            rows_v[r, pl.ds(0, L)] = rows_v[r, pl.ds(0, L)] + cl
            rows_v[r, pl.ds(L, L)] = rows_v[r, pl.ds(L, L)] + ch
            return carry2

        lax.fori_loop(0, CB, bias_r, 0)
        pltpu.sync_copy(rows_v, out.at[pl.ds(b0, CB), NNUM + j])
        return carry

    lax.fori_loop(0, NCAT, cat_j, 0)


def kernel(x_num, x_cat, W_num, b_num, table, cat_bias):
    xcT = x_cat.T
    return _emb(x_num.reshape(B * NNUM), xcT, W_num, b_num, table, cat_bias)
